# baseline (speedup 1.0000x reference)
import jax
import jax.numpy as jnp
from jax import lax
from jax.experimental import pallas as pl
from jax.experimental.pallas import tpu as pltpu
from jax.experimental.pallas import tpu_sc as plsc

N_ATOMS = 100000
N_GRAPHS = 1024
N_MODALS = 4
N_TYPES = 16

_L = 16
_NW = 32
_CHUNK = 3136
_TAIL = N_ATOMS - (_NW - 1) * _CHUNK
_TAIL_BASE = (_NW - 1) * _CHUNK


def _sc_body(energy_hbm, batch_hbm, modal_hbm, atype_hbm, shift_hbm,
             scale_hbm, out_hbm, energy_v, batch_v, atype_v, out_v,
             modal_v, shift_v, scale_v, sems):
    wid = lax.axis_index("s") * 2 + lax.axis_index("c")
    base = wid * _CHUNK

    table_copies = [
        pltpu.async_copy(modal_hbm, modal_v, sems.at[3]),
        pltpu.async_copy(shift_hbm, shift_v, sems.at[4]),
        pltpu.async_copy(scale_hbm, scale_v, sems.at[5]),
    ]

    @pl.when(wid < _NW - 1)
    def _():
        copies = [
            pltpu.async_copy(batch_hbm.at[pl.ds(base, _CHUNK)], batch_v,
                             sems.at[0]),
            pltpu.async_copy(atype_hbm.at[pl.ds(base, _CHUNK)], atype_v,
                             sems.at[1]),
            pltpu.async_copy(energy_hbm.at[pl.ds(base, _CHUNK)], energy_v,
                             sems.at[2]),
        ]
        for cp in copies:
            cp.wait()

    @pl.when(wid == _NW - 1)
    def _():
        copies = [
            pltpu.async_copy(batch_hbm.at[pl.ds(_TAIL_BASE, _TAIL)],
                             batch_v.at[pl.ds(0, _TAIL)], sems.at[0]),
            pltpu.async_copy(atype_hbm.at[pl.ds(_TAIL_BASE, _TAIL)],
                             atype_v.at[pl.ds(0, _TAIL)], sems.at[1]),
            pltpu.async_copy(energy_hbm.at[pl.ds(_TAIL_BASE, _TAIL)],
                             energy_v.at[pl.ds(0, _TAIL)], sems.at[2]),
        ]
        for cp in copies:
            cp.wait()
        zi = jnp.zeros((_L,), jnp.int32)
        for off in range(_TAIL, _CHUNK, _L):
            batch_v[pl.ds(off, _L)] = zi
            atype_v[pl.ds(off, _L)] = zi

    for cp in table_copies:
        cp.wait()

    @plsc.parallel_loop(0, _CHUNK, step=_L, unroll=16)
    def _(off):
        b = batch_v[pl.ds(off, _L)]
        t = atype_v[pl.ds(off, _L)]
        m = plsc.load_gather(modal_v, [b])
        c = plsc.load_gather(scale_v, [m, t])
        s = plsc.load_gather(shift_v, [m, t])
        e = energy_v[pl.ds(off, _L)]
        out_v[pl.ds(off, _L)] = e * c + s

    @pl.when(wid < _NW - 1)
    def _():
        pltpu.sync_copy(out_v, out_hbm.at[pl.ds(base, _CHUNK)])

    @pl.when(wid == _NW - 1)
    def _():
        pltpu.sync_copy(out_v.at[pl.ds(0, _TAIL)],
                        out_hbm.at[pl.ds(_TAIL_BASE, _TAIL)])


@jax.jit
def _rescale(energy, batch, modal_type, atom_type, shift, scale):
    mesh = plsc.VectorSubcoreMesh(core_axis_name="c", subcore_axis_name="s")
    run = pl.kernel(
        _sc_body,
        mesh=mesh,
        compiler_params=pltpu.CompilerParams(needs_layout_passes=False),
        out_type=jax.ShapeDtypeStruct((N_ATOMS,), jnp.float32),
        scratch_types=[
            pltpu.VMEM((_CHUNK,), jnp.float32),
            pltpu.VMEM((_CHUNK,), jnp.int32),
            pltpu.VMEM((_CHUNK,), jnp.int32),
            pltpu.VMEM((_CHUNK,), jnp.float32),
            pltpu.VMEM((N_GRAPHS,), jnp.int32),
            pltpu.VMEM((N_MODALS, N_TYPES), jnp.float32),
            pltpu.VMEM((N_MODALS, N_TYPES), jnp.float32),
            pltpu.SemaphoreType.DMA((6,)),
        ],
    )
    return run(energy, batch, modal_type, atom_type, shift, scale)


def kernel(scaled_atomic_energy, batch, modal_type, atom_type, shift, scale):
    out = _rescale(scaled_atomic_energy.reshape(-1),
                   batch.astype(jnp.int32),
                   modal_type.astype(jnp.int32),
                   atom_type.astype(jnp.int32),
                   shift, scale)
    return out.reshape(-1, 1)


# speedup vs baseline: 1.0101x; 1.0101x over previous
import jax
import jax.numpy as jnp
from jax import lax
from jax.experimental import pallas as pl
from jax.experimental.pallas import tpu as pltpu
from jax.experimental.pallas import tpu_sc as plsc

N_ATOMS = 100000
N_GRAPHS = 1024
N_MODALS = 4
N_TYPES = 16

_L = 16
_NW = 32
_CHUNK = 3136
_TAIL = N_ATOMS - (_NW - 1) * _CHUNK
_TAIL_BASE = (_NW - 1) * _CHUNK


def _sc_body(energy_hbm, batch_hbm, modal_hbm, atype_hbm, shift_hbm,
             scale_hbm, out_hbm, energy_v, batch_v, atype_v, out_v,
             modal_v, shift_v, scale_v, sems):
    wid = lax.axis_index("s") * 2 + lax.axis_index("c")
    base = wid * _CHUNK

    table_copies = [
        pltpu.async_copy(modal_hbm, modal_v, sems.at[3]),
        pltpu.async_copy(shift_hbm, shift_v, sems.at[4]),
        pltpu.async_copy(scale_hbm, scale_v, sems.at[5]),
    ]

    @pl.when(wid < _NW - 1)
    def _():
        copies = [
            pltpu.async_copy(batch_hbm.at[pl.ds(base, _CHUNK)], batch_v,
                             sems.at[0]),
            pltpu.async_copy(atype_hbm.at[pl.ds(base, _CHUNK)], atype_v,
                             sems.at[1]),
            pltpu.async_copy(energy_hbm.at[pl.ds(base, _CHUNK)], energy_v,
                             sems.at[2]),
        ]
        for cp in copies:
            cp.wait()

    @pl.when(wid == _NW - 1)
    def _():
        copies = [
            pltpu.async_copy(batch_hbm.at[pl.ds(_TAIL_BASE, _TAIL)],
                             batch_v.at[pl.ds(0, _TAIL)], sems.at[0]),
            pltpu.async_copy(atype_hbm.at[pl.ds(_TAIL_BASE, _TAIL)],
                             atype_v.at[pl.ds(0, _TAIL)], sems.at[1]),
            pltpu.async_copy(energy_hbm.at[pl.ds(_TAIL_BASE, _TAIL)],
                             energy_v.at[pl.ds(0, _TAIL)], sems.at[2]),
        ]
        for cp in copies:
            cp.wait()
        zi = jnp.zeros((_L,), jnp.int32)
        for off in range(_TAIL, _CHUNK, _L):
            batch_v[pl.ds(off, _L)] = zi
            atype_v[pl.ds(off, _L)] = zi

    for cp in table_copies:
        cp.wait()

    @plsc.parallel_loop(0, _CHUNK, step=_L, unroll=8)
    def _(off):
        b = batch_v[pl.ds(off, _L)]
        t = atype_v[pl.ds(off, _L)]
        m = plsc.load_gather(modal_v, [b])
        c = plsc.load_gather(scale_v, [m, t])
        s = plsc.load_gather(shift_v, [m, t])
        e = energy_v[pl.ds(off, _L)]
        out_v[pl.ds(off, _L)] = e * c + s

    @pl.when(wid < _NW - 1)
    def _():
        pltpu.sync_copy(out_v, out_hbm.at[pl.ds(base, _CHUNK)])

    @pl.when(wid == _NW - 1)
    def _():
        pltpu.sync_copy(out_v.at[pl.ds(0, _TAIL)],
                        out_hbm.at[pl.ds(_TAIL_BASE, _TAIL)])


@jax.jit
def _rescale(energy, batch, modal_type, atom_type, shift, scale):
    mesh = plsc.VectorSubcoreMesh(core_axis_name="c", subcore_axis_name="s")
    run = pl.kernel(
        _sc_body,
        mesh=mesh,
        compiler_params=pltpu.CompilerParams(needs_layout_passes=False),
        out_type=jax.ShapeDtypeStruct((N_ATOMS,), jnp.float32),
        scratch_types=[
            pltpu.VMEM((_CHUNK,), jnp.float32),
            pltpu.VMEM((_CHUNK,), jnp.int32),
            pltpu.VMEM((_CHUNK,), jnp.int32),
            pltpu.VMEM((_CHUNK,), jnp.float32),
            pltpu.VMEM((N_GRAPHS,), jnp.int32),
            pltpu.VMEM((N_MODALS, N_TYPES), jnp.float32),
            pltpu.VMEM((N_MODALS, N_TYPES), jnp.float32),
            pltpu.SemaphoreType.DMA((6,)),
        ],
    )
    return run(energy, batch, modal_type, atom_type, shift, scale)


def kernel(scaled_atomic_energy, batch, modal_type, atom_type, shift, scale):
    out = _rescale(scaled_atomic_energy.reshape(-1),
                   batch.astype(jnp.int32),
                   modal_type.astype(jnp.int32),
                   atom_type.astype(jnp.int32),
                   shift, scale)
    return out.reshape(-1, 1)
